# baseline (speedup 1.0000x reference)
import functools

import jax
import jax.numpy as jnp
from jax.experimental import pallas as pl

N1 = 10000
N2 = 10000
HEADS = 4


def _prep_edges(ei1, ei2, n1, n2):
    ei1 = ei1.astype(jnp.int32)
    ei2 = ei2.astype(jnp.int32)
    N = n1 + n2
    ei2o = ei2 + n1
    ei1o = ei1 + n1
    valid = (ei1o[0] < N) & (ei1o[1] < N)
    comb = jnp.concatenate([ei1, ei2o, ei1o], axis=1)
    valid_all = jnp.concatenate(
        [jnp.ones((2 * ei1.shape[1],), dtype=bool), valid]
    )
    key = comb[0] * N + comb[1]
    sentinel = N * N
    key = jnp.where(valid_all, key, sentinel)
    key = jnp.sort(key)
    first = jnp.concatenate([jnp.ones((1,), dtype=bool), key[1:] != key[:-1]])
    row = key // N
    col = key % N
    keep = first & (key < sentinel) & (row != col)
    srcm = jnp.where(keep, row, 0)
    dstm = jnp.where(keep, col, N)
    ar = jnp.arange(N, dtype=jnp.int32)
    src = jnp.concatenate([srcm.astype(jnp.int32), ar])
    dst = jnp.concatenate([dstm.astype(jnp.int32), ar])
    return src, dst


def _gat(x, src, dst, W, a_s, a_d, b, N):
    H, C = a_s.shape
    xl = (x @ W).reshape(N, H, C)
    a_src = (xl * a_s[None, :, :]).sum(-1)
    a_dst = (xl * a_d[None, :, :]).sum(-1)
    alpha = jax.nn.leaky_relu(a_src[src] + a_dst[dst], 0.2)
    amax = jax.ops.segment_max(alpha, dst, num_segments=N)
    amax = jnp.where(jnp.isfinite(amax), amax, 0.0)
    ex = jnp.exp(alpha - amax[dst])
    den = jax.ops.segment_sum(ex, dst, num_segments=N) + 1e-16
    att = ex / den[dst]
    out = jax.ops.segment_sum(xl[src] * att[:, :, None], dst, num_segments=N)
    return out.mean(axis=1) + b


def _final_kernel(h2_ref, res_ref, out_ref):
    h2 = h2_ref[...]
    nrm = jnp.sqrt(jnp.sum(h2 * h2, axis=1, keepdims=True))
    out_ref[...] = h2 / jnp.maximum(nrm, 1e-12) + res_ref[...]


def kernel(x1, x2, edge_index1, edge_index2, params):
    N = N1 + N2
    src, dst = _prep_edges(edge_index1, edge_index2, N1, N2)
    xc = jnp.concatenate([x1, x2], axis=0)
    residual = x2 @ params['Wres'] + params['bres']
    h = xc
    h = jax.nn.relu(_gat(h, src, dst, params['W0'], params['as0'], params['ad0'], params['b0'], N))
    h = jax.nn.relu(_gat(h, src, dst, params['W1'], params['as1'], params['ad1'], params['b1'], N))
    h = jax.nn.relu(_gat(h, src, dst, params['W2'], params['as2'], params['ad2'], params['b2'], N))
    h = _gat(h, src, dst, params['Wf'], params['asf'], params['adf'], params['bf'], N)
    h2 = h[N1:N1 + N2]
    out = pl.pallas_call(
        _final_kernel,
        out_shape=jax.ShapeDtypeStruct((N2, h2.shape[1]), jnp.float32),
    )(h2, residual)
    return out


# speedup vs baseline: 13.0974x; 13.0974x over previous
import functools

import jax
import jax.numpy as jnp
from jax import lax
from jax.experimental import pallas as pl
from jax.experimental.pallas import tpu as pltpu
from jax.experimental.pallas import tpu_sc as plsc

N1 = 10000
N2 = 10000
NTOT = N1 + N2
IN_CH = 128
HID = 64
HEADS = 4

NPAD = 20480
QRT = NPAD // 4
SENT = NTOT * NTOT
ERAW = 3 * 160000 + NTOT
EPAD = 524288
KC = 128
ACCR = QRT + 128
STRIPE = ACCR // 16
ZR = 8
DENR = QRT + 16
NEG = -1e30


def _edge_prep(ei1, ei2):
    ei1 = ei1.astype(jnp.int32)
    ei2 = ei2.astype(jnp.int32)
    src3 = jnp.concatenate([ei1[0], ei2[0] + N1, ei1[0] + N1])
    dst3 = jnp.concatenate([ei1[1], ei2[1] + N1, ei1[1] + N1])
    key = dst3 * NTOT + src3
    key = jnp.where(src3 == dst3, SENT, key)
    ar = jnp.arange(NTOT, dtype=jnp.int32)
    keyd = ar * NTOT + ar
    keys = jnp.sort(jnp.concatenate([key, keyd]))
    dup = jnp.concatenate(
        [jnp.zeros((1,), bool), keys[1:] == keys[:-1]])
    dst = jnp.minimum(keys // NTOT, NTOT)
    src = jnp.where(dup | (keys >= SENT), NTOT, keys % NTOT)
    pad = EPAD - ERAW
    src = jnp.concatenate([src, jnp.full((pad,), NTOT, jnp.int32)])
    dst = jnp.concatenate([dst, jnp.full((pad,), NTOT, jnp.int32)])
    los = [jnp.searchsorted(dst, jnp.int32(q * QRT)).astype(jnp.int32)
           for q in range(1, 4)] + [jnp.int32(EPAD)]
    ent = []
    prev_lo = jnp.int32(0)
    for q in range(4):
        lor = (prev_lo // KC) * KC
        nch = (los[q] - lor + KC - 1) // KC
        ent += [lor, nch]
        prev_lo = los[q]
    bnds = jnp.concatenate([jnp.stack(ent), jnp.zeros((8,), jnp.int32)])
    return src, dst, bnds


def _pre_body(has_acc, np_prev, c_prev, npair, c_out, *refs):
    if has_acc:
        (acc_ref, dnp_ref, b_ref, w_ref, as_ref, ad_ref,
         xl_ref, asr_ref, adr_ref, gm_ref) = refs
        den_s = jnp.sum(dnp_ref[...], axis=2)
        parts = []
        for p in range(c_prev // 64):
            num = jnp.zeros((512, 64), jnp.float32)
            for h in range(HEADS):
                j0 = h * c_prev + p * 64
                pair, colo = j0 // 128, j0 % 128
                sub = h % 2 if c_prev == 64 else 0
                den = den_s[pair, sub, :].reshape(512, 1) + 1e-16
                num = num + acc_ref[pair, :, colo:colo + 64] / den
            parts.append(num * (1.0 / HEADS) + b_ref[0, p * 64:(p + 1) * 64])
        xb = jnp.concatenate(parts, axis=1) if len(parts) > 1 else parts[0]
        xb = jnp.maximum(xb, 0.0)
    else:
        (x_ref, w_ref, as_ref, ad_ref,
         xl_ref, asr_ref, adr_ref, gm_ref) = refs
        xb = x_ref[...]
    g = pl.program_id(0)
    y = jnp.dot(xb, w_ref[...], preferred_element_type=jnp.float32)
    rows = lax.broadcasted_iota(jnp.int32, (512, 1), 0) + g * 512
    mask = rows < NTOT
    as_cols, ad_cols, gmx = [], [], []
    for h in range(HEADS):
        yh = y[:, h * c_out:(h + 1) * c_out]
        a_s = jnp.sum(yh * as_ref[h, :][None, :], axis=1, keepdims=True)
        a_d = jnp.sum(yh * ad_ref[h, :][None, :], axis=1, keepdims=True)
        a_s = jnp.where(mask, a_s, NEG)
        a_d = jnp.where(mask, a_d, NEG)
        as_cols.append(a_s)
        ad_cols.append(a_d)
        gmx.append(jnp.max(a_s))
    padc = jnp.full((512, 16 - HEADS), NEG)
    asr_ref[...] = jnp.concatenate(as_cols + [padc], axis=1)
    adr_ref[...] = jnp.concatenate(ad_cols + [padc], axis=1)
    m4 = jnp.stack(gmx).reshape(1, HEADS)
    prev = jnp.where(g == 0, jnp.full((1, HEADS), NEG), gm_ref[...])
    gm_ref[...] = jnp.maximum(prev, m4)
    for q in range(npair):
        xl_ref[q] = y[:, q * 128:(q + 1) * 128]


def _pre_call(has_acc, np_prev, c_prev, c_out, first_arg, w, a_s, a_d,
              bias=None, denp=None):
    npair = HEADS * c_out // 128
    grid = NPAD // 512
    body = functools.partial(_pre_body, has_acc, np_prev, c_prev,
                             npair, c_out)
    in_specs = []
    args = []
    if has_acc:
        in_specs.append(pl.BlockSpec((np_prev, 512, 128),
                                     lambda g: (0, g, 0)))
        args.append(first_arg)
        in_specs.append(pl.BlockSpec((np_prev, 2, 16, 512),
                                     lambda g: (0, 0, 0, g)))
        args.append(denp)
        in_specs.append(pl.BlockSpec((1, c_prev), lambda g: (0, 0)))
        args.append(bias.reshape(1, c_prev))
    else:
        in_specs.append(pl.BlockSpec((512, c_prev), lambda g: (g, 0)))
        args.append(first_arg)
    in_specs += [
        pl.BlockSpec((c_prev, HEADS * c_out), lambda g: (0, 0)),
        pl.BlockSpec((HEADS, c_out), lambda g: (0, 0)),
        pl.BlockSpec((HEADS, c_out), lambda g: (0, 0)),
    ]
    args += [w, a_s, a_d]
    return pl.pallas_call(
        body,
        grid=(grid,),
        in_specs=in_specs,
        out_specs=[
            pl.BlockSpec((npair, 512, 128), lambda g: (0, g, 0)),
            pl.BlockSpec((512, 16), lambda g: (g, 0)),
            pl.BlockSpec((512, 16), lambda g: (g, 0)),
            pl.BlockSpec((1, HEADS), lambda g: (0, 0)),
        ],
        out_shape=[
            jax.ShapeDtypeStruct((npair, NPAD, 128), jnp.float32),
            jax.ShapeDtypeStruct((NPAD, 16), jnp.float32),
            jax.ShapeDtypeStruct((NPAD, 16), jnp.float32),
            jax.ShapeDtypeStruct((1, HEADS), jnp.float32),
        ],
    )(*args)


def _vextract(vec, j):
    return jnp.max(jnp.where(lax.iota(jnp.int32, 16) == j, vec, 0))


@functools.lru_cache(maxsize=None)
def _make_edge_kernel(npair, c_out):
    two_heads = (c_out == 64)

    def body(xlp, ash, adh, gmx, src_e, dst_e, bnds, acc2, denp,
             asv0, adv0, asv1, adv1, srcv, dstv, ldv, rb,
             ex0, ex1, den0, den1, zb, gm0v, gm1v, bndv, sem, acc_sp):
        c = lax.axis_index("c")
        s = lax.axis_index("s")
        zero16 = jnp.zeros((16,), jnp.float32)

        def zrow(r, carry):
            for q in range(8):
                zb[r, pl.ds(q * 16, 16)] = zero16
            return carry
        lax.fori_loop(0, ZR, zrow, 0)

        pltpu.sync_copy(bnds, bndv)

        for i in range(npair):
            h0 = (i * 128) // c_out
            h1 = (i * 128 + 64) // c_out
            pltpu.sync_copy(gmx.at[pl.ds(h0 * 16, 16)], gm0v)
            pltpu.sync_copy(ash.at[pl.ds(h0 * NPAD, NPAD)], asv0)
            if two_heads:
                pltpu.sync_copy(gmx.at[pl.ds(h1 * 16, 16)], gm1v)
                pltpu.sync_copy(ash.at[pl.ds(h1 * NPAD, NPAD)], asv1)
            gm0 = gm0v[...]
            gm1 = gm1v[...] if two_heads else gm0
            row_off = i * NPAD

            for qq in range(2):
                quarter = qq * 2 + c
                base = pl.multiple_of(quarter * QRT, QRT)
                lo = pl.multiple_of(
                    _vextract(bndv[...], 2 * quarter), KC)
                nch = _vextract(bndv[...], 2 * quarter + 1)
                cnt_s = jnp.maximum(0, (nch - s + 15) // 16)
                pltpu.sync_copy(
                    adh.at[pl.ds(pl.multiple_of(h0 * NPAD + base, QRT),
                                 QRT)], adv0)
                if two_heads:
                    pltpu.sync_copy(
                        adh.at[pl.ds(pl.multiple_of(h1 * NPAD + base, QRT),
                                     QRT)], adv1)
                for z in range(STRIPE // ZR):
                    pltpu.sync_copy(
                        zb, acc_sp.at[pl.ds(s * STRIPE + z * ZR, ZR)])

                def zden(r, carry):
                    den0[pl.ds(r * 16, 16)] = zero16
                    if two_heads:
                        den1[pl.ds(r * 16, 16)] = zero16
                    return carry
                lax.fori_loop(0, DENR // 16, zden, 0)
                plsc.subcore_barrier()

                def chunk(ii, carry):
                    off = pl.multiple_of(lo + (s + ii * 16) * KC, KC)
                    pltpu.sync_copy(src_e.at[pl.ds(off, KC)], srcv)
                    pltpu.sync_copy(dst_e.at[pl.ds(off, KC)], dstv)
                    for g in range(8):
                        sv = srcv[pl.ds(g * 16, 16)]
                        dv = dstv[pl.ds(g * 16, 16)]
                        ld = dv - base
                        inb = jnp.logical_and(ld >= 0, ld < QRT)
                        ldg = jnp.minimum(jnp.maximum(ld, 0), QRT - 1)
                        ldc = jnp.where(inb, ld, QRT)
                        ldv[pl.ds(g * 16, 16)] = ldc
                        dgat = plsc.load_gather(adv0, [ldg])
                        pre = plsc.load_gather(asv0, [sv]) + dgat
                        alpha = jnp.maximum(pre, 0.2 * pre)
                        m = gm0 + dgat
                        e0 = jnp.exp(alpha - jnp.maximum(m, 0.2 * m))
                        ex0[pl.ds(g * 16, 16)] = e0
                        plsc.addupdate_scatter(den0, [ldc], e0)
                        if two_heads:
                            dgat1 = plsc.load_gather(adv1, [ldg])
                            pre1 = plsc.load_gather(asv1, [sv]) + dgat1
                            alpha1 = jnp.maximum(pre1, 0.2 * pre1)
                            m1 = gm1 + dgat1
                            e1 = jnp.exp(alpha1 - jnp.maximum(m1, 0.2 * m1))
                            ex1[pl.ds(g * 16, 16)] = e1
                            plsc.addupdate_scatter(den1, [ldc], e1)
                        srcv[pl.ds(g * 16, 16)] = sv + row_off
                    pltpu.async_copy(xlp.at[srcv], rb, sem).wait()

                    def krow(k, kc):
                        k16 = jnp.full((16,), 0, jnp.int32) + k
                        b0 = plsc.load_gather(ex0, [k16])
                        b1 = (plsc.load_gather(ex1, [k16])
                              if two_heads else b0)
                        for q in range(4):
                            rb[k, pl.ds(q * 16, 16)] = (
                                rb[k, pl.ds(q * 16, 16)] * b0)
                        for q in range(4, 8):
                            rb[k, pl.ds(q * 16, 16)] = (
                                rb[k, pl.ds(q * 16, 16)] * b1)
                        return kc
                    lax.fori_loop(0, KC, krow, 0)
                    pltpu.sync_copy(rb, acc_sp.at[ldv], add=True)
                    return carry
                lax.fori_loop(0, cnt_s, chunk, 0)
                plsc.subcore_barrier()
                drw = QRT // 16
                pltpu.sync_copy(
                    acc_sp.at[pl.ds(pl.multiple_of(s * drw, drw), drw)],
                    acc2.at[pl.ds(
                        pl.multiple_of(row_off + base + s * drw, drw), drw)])
                pltpu.sync_copy(
                    den0.at[pl.ds(0, QRT)],
                    denp.at[pl.ds(pl.multiple_of(
                        ((i * 2) * 16 + s) * NPAD + base, QRT), QRT)])
                if two_heads:
                    pltpu.sync_copy(
                        den1.at[pl.ds(0, QRT)],
                        denp.at[pl.ds(pl.multiple_of(
                            ((i * 2 + 1) * 16 + s) * NPAD + base, QRT),
                            QRT)])
                plsc.subcore_barrier()

    mesh = plsc.VectorSubcoreMesh(core_axis_name="c", subcore_axis_name="s")
    scratch = [
        pltpu.VMEM((NPAD,), jnp.float32),
        pltpu.VMEM((QRT,), jnp.float32),
        pltpu.VMEM((NPAD,), jnp.float32),
        pltpu.VMEM((QRT,), jnp.float32),
        pltpu.VMEM((KC,), jnp.int32),
        pltpu.VMEM((KC,), jnp.int32),
        pltpu.VMEM((KC,), jnp.int32),
        pltpu.VMEM((KC, 128), jnp.float32),
        pltpu.VMEM((KC,), jnp.float32),
        pltpu.VMEM((KC,), jnp.float32),
        pltpu.VMEM((DENR,), jnp.float32),
        pltpu.VMEM((DENR,), jnp.float32),
        pltpu.VMEM((ZR, 128), jnp.float32),
        pltpu.VMEM((16,), jnp.float32),
        pltpu.VMEM((16,), jnp.float32),
        pltpu.VMEM((16,), jnp.int32),
        pltpu.SemaphoreType.DMA,
        pltpu.VMEM_SHARED((ACCR, 128), jnp.float32),
    ]

    return pl.kernel(
        body,
        out_type=(
            jax.ShapeDtypeStruct((npair * NPAD, 128), jnp.float32),
            jax.ShapeDtypeStruct((npair * 2 * 16 * NPAD,), jnp.float32),
        ),
        mesh=mesh,
        scratch_types=scratch,
        compiler_params=pltpu.CompilerParams(needs_layout_passes=False),
    )


def _final_body(acc_ref, dnp_ref, x2_ref, wr_ref, br_ref, bf_ref, out_ref):
    den_s = jnp.sum(dnp_ref[...], axis=2)
    parts = []
    for p in range(2):
        num = jnp.zeros((512, 64), jnp.float32)
        for h in range(HEADS):
            den = den_s[h, 0, :].reshape(512, 1) + 1e-16
            num = num + acc_ref[h, :, p * 64:(p + 1) * 64] / den
        parts.append(num * (1.0 / HEADS) + bf_ref[0, p * 64:(p + 1) * 64])
    h2 = jnp.concatenate(parts, axis=1)
    nrm = jnp.sqrt(jnp.sum(h2 * h2, axis=1, keepdims=True))
    h2 = h2 / jnp.maximum(nrm, 1e-12)
    res = jnp.dot(x2_ref[...], wr_ref[...],
                  preferred_element_type=jnp.float32) + br_ref[0, :][None, :]
    out_ref[...] = h2 + res


def _final_call(accf, denpf, x2, wres, bres, bf):
    x2p = jnp.concatenate(
        [jnp.zeros((N1, IN_CH), jnp.float32), x2,
         jnp.zeros((NPAD - NTOT, IN_CH), jnp.float32)], axis=0)
    full = pl.pallas_call(
        _final_body,
        grid=(NPAD // 512,),
        in_specs=[
            pl.BlockSpec((4, 512, 128), lambda g: (0, g, 0)),
            pl.BlockSpec((4, 2, 16, 512), lambda g: (0, 0, 0, g)),
            pl.BlockSpec((512, IN_CH), lambda g: (g, 0)),
            pl.BlockSpec((IN_CH, IN_CH), lambda g: (0, 0)),
            pl.BlockSpec((1, IN_CH), lambda g: (0, 0)),
            pl.BlockSpec((1, IN_CH), lambda g: (0, 0)),
        ],
        out_specs=pl.BlockSpec((512, IN_CH), lambda g: (g, 0)),
        out_shape=jax.ShapeDtypeStruct((NPAD, IN_CH), jnp.float32),
    )(accf, denpf, x2p, wres, bres.reshape(1, IN_CH), bf.reshape(1, IN_CH))
    return full[N1:NTOT]


def _layer_sc(c_out, xl, ash, adh, gm, src_e, dst_e, bnds):
    npair = HEADS * c_out // 128
    gmx = jnp.broadcast_to(gm.reshape(HEADS, 1), (HEADS, 16)).reshape(-1)
    ash_t = ash.T[:HEADS].reshape(-1)
    adh_t = adh.T[:HEADS].reshape(-1)
    acc2, denp = _make_edge_kernel(npair, c_out)(
        xl.reshape(npair * NPAD, 128), ash_t, adh_t, gmx,
        src_e, dst_e, bnds)
    return (acc2.reshape(npair, NPAD, 128),
            denp.reshape(npair, 2, 16, NPAD))


def kernel(x1, x2, edge_index1, edge_index2, params):
    p = params
    src_e, dst_e, bnds = _edge_prep(edge_index1, edge_index2)
    xc = jnp.concatenate(
        [x1, x2, jnp.zeros((NPAD - NTOT, IN_CH), jnp.float32)], axis=0)

    xl, ash, adh, gm = _pre_call(False, 0, IN_CH, HID, xc,
                                 p['W0'], p['as0'], p['ad0'])
    acc, denp = _layer_sc(HID, xl, ash, adh, gm, src_e, dst_e, bnds)

    xl, ash, adh, gm = _pre_call(True, 2, HID, HID, acc, p['W1'],
                                 p['as1'], p['ad1'], bias=p['b0'], denp=denp)
    acc, denp = _layer_sc(HID, xl, ash, adh, gm, src_e, dst_e, bnds)

    xl, ash, adh, gm = _pre_call(True, 2, HID, IN_CH, acc, p['W2'],
                                 p['as2'], p['ad2'], bias=p['b1'], denp=denp)
    acc, denp = _layer_sc(IN_CH, xl, ash, adh, gm, src_e, dst_e, bnds)

    xl, ash, adh, gm = _pre_call(True, 4, IN_CH, IN_CH, acc, p['Wf'],
                                 p['asf'], p['adf'], bias=p['b2'], denp=denp)
    acc, denp = _layer_sc(IN_CH, xl, ash, adh, gm, src_e, dst_e, bnds)

    return _final_call(acc, denp, x2, p['Wres'], p['bres'], p['bf'])


# speedup vs baseline: 14.5522x; 1.1111x over previous
import functools

import jax
import jax.numpy as jnp
from jax import lax
from jax.experimental import pallas as pl
from jax.experimental.pallas import tpu as pltpu
from jax.experimental.pallas import tpu_sc as plsc

N1 = 10000
N2 = 10000
NTOT = N1 + N2
IN_CH = 128
HID = 64
HEADS = 4

NPAD = 20480
QRT = NPAD // 4
SENT = NTOT * NTOT
ERAW = 3 * 160000 + NTOT
EPAD = 524288
KC = 128
ACCR = QRT + 128
STRIPE = ACCR // 16
ZR = 8
DENR = QRT + 16
NEG = -1e30


def _edge_prep(ei1, ei2):
    ei1 = ei1.astype(jnp.int32)
    ei2 = ei2.astype(jnp.int32)
    src3 = jnp.concatenate([ei1[0], ei2[0] + N1, ei1[0] + N1])
    dst3 = jnp.concatenate([ei1[1], ei2[1] + N1, ei1[1] + N1])
    key = dst3 * NTOT + src3
    key = jnp.where(src3 == dst3, SENT, key)
    ar = jnp.arange(NTOT, dtype=jnp.int32)
    keyd = ar * NTOT + ar
    keys = jnp.sort(jnp.concatenate([key, keyd]))
    dup = jnp.concatenate(
        [jnp.zeros((1,), bool), keys[1:] == keys[:-1]])
    dst = jnp.minimum(keys // NTOT, NTOT)
    src = jnp.where(dup | (keys >= SENT), NTOT, keys % NTOT)
    pad = EPAD + 3072 - ERAW
    src = jnp.concatenate([src, jnp.full((pad,), NTOT, jnp.int32)])
    dst = jnp.concatenate([dst, jnp.full((pad,), NTOT, jnp.int32)])
    los = [jnp.searchsorted(dst, jnp.int32(q * QRT)).astype(jnp.int32)
           for q in range(1, 4)] + [jnp.int32(EPAD)]
    ent = []
    prev_lo = jnp.int32(0)
    for q in range(4):
        lor = (prev_lo // KC) * KC
        nch = (los[q] - lor + KC - 1) // KC
        ent += [lor, nch]
        prev_lo = los[q]
    bnds = jnp.concatenate([jnp.stack(ent), jnp.zeros((8,), jnp.int32)])
    return src, dst, bnds


def _pre_body(has_acc, np_prev, c_prev, npair, c_out, *refs):
    if has_acc:
        (acc_ref, dnp_ref, b_ref, w_ref, as_ref, ad_ref,
         xl_ref, asr_ref, adr_ref, gm_ref) = refs
        den_s = jnp.sum(dnp_ref[...], axis=2)
        parts = []
        for p in range(c_prev // 64):
            num = jnp.zeros((512, 64), jnp.float32)
            for h in range(HEADS):
                j0 = h * c_prev + p * 64
                pair, colo = j0 // 128, j0 % 128
                sub = h % 2 if c_prev == 64 else 0
                den = den_s[pair, sub, :].reshape(512, 1) + 1e-16
                num = num + acc_ref[pair, :, colo:colo + 64] / den
            parts.append(num * (1.0 / HEADS) + b_ref[0, p * 64:(p + 1) * 64])
        xb = jnp.concatenate(parts, axis=1) if len(parts) > 1 else parts[0]
        xb = jnp.maximum(xb, 0.0)
    else:
        (x_ref, w_ref, as_ref, ad_ref,
         xl_ref, asr_ref, adr_ref, gm_ref) = refs
        xb = x_ref[...]
    g = pl.program_id(0)
    y = jnp.dot(xb, w_ref[...], preferred_element_type=jnp.float32)
    rows = lax.broadcasted_iota(jnp.int32, (512, 1), 0) + g * 512
    mask = rows < NTOT
    as_cols, ad_cols, gmx = [], [], []
    for h in range(HEADS):
        yh = y[:, h * c_out:(h + 1) * c_out]
        a_s = jnp.sum(yh * as_ref[h, :][None, :], axis=1, keepdims=True)
        a_d = jnp.sum(yh * ad_ref[h, :][None, :], axis=1, keepdims=True)
        a_s = jnp.where(mask, a_s, NEG)
        a_d = jnp.where(mask, a_d, NEG)
        as_cols.append(a_s)
        ad_cols.append(a_d)
        gmx.append(jnp.max(a_s))
    padc = jnp.full((512, 16 - HEADS), NEG)
    asr_ref[...] = jnp.concatenate(as_cols + [padc], axis=1)
    adr_ref[...] = jnp.concatenate(ad_cols + [padc], axis=1)
    m4 = jnp.stack(gmx).reshape(1, HEADS)
    prev = jnp.where(g == 0, jnp.full((1, HEADS), NEG), gm_ref[...])
    gm_ref[...] = jnp.maximum(prev, m4)
    for q in range(npair):
        xl_ref[q] = y[:, q * 128:(q + 1) * 128]


def _pre_call(has_acc, np_prev, c_prev, c_out, first_arg, w, a_s, a_d,
              bias=None, denp=None):
    npair = HEADS * c_out // 128
    grid = NPAD // 512
    body = functools.partial(_pre_body, has_acc, np_prev, c_prev,
                             npair, c_out)
    in_specs = []
    args = []
    if has_acc:
        in_specs.append(pl.BlockSpec((np_prev, 512, 128),
                                     lambda g: (0, g, 0)))
        args.append(first_arg)
        in_specs.append(pl.BlockSpec((np_prev, 2, 16, 512),
                                     lambda g: (0, 0, 0, g)))
        args.append(denp)
        in_specs.append(pl.BlockSpec((1, c_prev), lambda g: (0, 0)))
        args.append(bias.reshape(1, c_prev))
    else:
        in_specs.append(pl.BlockSpec((512, c_prev), lambda g: (g, 0)))
        args.append(first_arg)
    in_specs += [
        pl.BlockSpec((c_prev, HEADS * c_out), lambda g: (0, 0)),
        pl.BlockSpec((HEADS, c_out), lambda g: (0, 0)),
        pl.BlockSpec((HEADS, c_out), lambda g: (0, 0)),
    ]
    args += [w, a_s, a_d]
    return pl.pallas_call(
        body,
        grid=(grid,),
        in_specs=in_specs,
        out_specs=[
            pl.BlockSpec((npair, 512, 128), lambda g: (0, g, 0)),
            pl.BlockSpec((512, 16), lambda g: (g, 0)),
            pl.BlockSpec((512, 16), lambda g: (g, 0)),
            pl.BlockSpec((1, HEADS), lambda g: (0, 0)),
        ],
        out_shape=[
            jax.ShapeDtypeStruct((npair, NPAD, 128), jnp.float32),
            jax.ShapeDtypeStruct((NPAD, 16), jnp.float32),
            jax.ShapeDtypeStruct((NPAD, 16), jnp.float32),
            jax.ShapeDtypeStruct((1, HEADS), jnp.float32),
        ],
    )(*args)


def _vextract(vec, j):
    return jnp.max(jnp.where(lax.iota(jnp.int32, 16) == j, vec, 0))


@functools.lru_cache(maxsize=None)
def _make_edge_kernel(npair, c_out):
    two_heads = (c_out == 64)
    kcl = 64 if two_heads else 128
    ng = kcl // 16

    def body(xlp, ash, adh, gmx, src_e, dst_e, bnds, acc2, denp, *scr):
        if two_heads:
            (asv0, adv0, asv1, adv1, srcvb, dstvb, srcg0, srcg1,
             ldv0, ldv1, rb0, rb1, ex0a, ex0b, ex1a, ex1b,
             den0, den1, zb, gm0v, gm1v, bndv, sem0, sem1, acc_sp) = scr
        else:
            (asv0, adv0, srcvb, dstvb, srcg0, srcg1,
             ldv0, ldv1, rb0, rb1, ex0a, ex0b,
             den0, zb, gm0v, bndv, sem0, sem1, acc_sp) = scr
            asv1 = adv1 = den1 = gm1v = None
            ex1a = ex1b = None
        srcg_r = (srcg0, srcg1)
        ldv_r = (ldv0, ldv1)
        rb_r = (rb0, rb1)
        ex0_r = (ex0a, ex0b)
        ex1_r = (ex1a, ex1b)
        sem_r = (sem0, sem1)
        c = lax.axis_index("c")
        s = lax.axis_index("s")
        zero16 = jnp.zeros((16,), jnp.float32)

        def zrow(r, carry):
            for q in range(8):
                zb[r, pl.ds(q * 16, 16)] = zero16
            return carry
        lax.fori_loop(0, ZR, zrow, 0)

        pltpu.sync_copy(bnds, bndv)

        for i in range(npair):
            h0 = (i * 128) // c_out
            h1 = (i * 128 + 64) // c_out
            pltpu.sync_copy(gmx.at[pl.ds(h0 * 16, 16)], gm0v)
            pltpu.sync_copy(ash.at[pl.ds(h0 * NPAD, NPAD)], asv0)
            if two_heads:
                pltpu.sync_copy(gmx.at[pl.ds(h1 * 16, 16)], gm1v)
                pltpu.sync_copy(ash.at[pl.ds(h1 * NPAD, NPAD)], asv1)
            gm0 = gm0v[...]
            gm1 = gm1v[...] if two_heads else gm0
            row_off = i * NPAD

            for qq in range(2):
                quarter = qq * 2 + c
                base = pl.multiple_of(quarter * QRT, QRT)
                lo = pl.multiple_of(
                    _vextract(bndv[...], 2 * quarter), KC)
                nchl = _vextract(bndv[...], 2 * quarter + 1) * (KC // kcl)
                per = (nchl + 15) // 16
                cnt_s = jnp.maximum(0, jnp.minimum(per, nchl - s * per))
                jcnt = cnt_s // 8 + 1
                pltpu.sync_copy(
                    adh.at[pl.ds(pl.multiple_of(h0 * NPAD + base, QRT),
                                 QRT)], adv0)
                if two_heads:
                    pltpu.sync_copy(
                        adh.at[pl.ds(pl.multiple_of(h1 * NPAD + base, QRT),
                                     QRT)], adv1)
                for z in range(STRIPE // ZR):
                    pltpu.sync_copy(
                        zb, acc_sp.at[pl.ds(s * STRIPE + z * ZR, ZR)])

                def zden(r, carry):
                    den0[pl.ds(r * 16, 16)] = zero16
                    if two_heads:
                        den1[pl.ds(r * 16, 16)] = zero16
                    return carry
                lax.fori_loop(0, DENR // 16, zden, 0)
                plsc.subcore_barrier()

                def stage(bb, p):
                    for g in range(ng):
                        sv = srcvb[pl.ds(bb * kcl + g * 16, 16)]
                        dv = dstvb[pl.ds(bb * kcl + g * 16, 16)]
                        ld = dv - base
                        inb = jnp.logical_and(ld >= 0, ld < QRT)
                        ldg = jnp.minimum(jnp.maximum(ld, 0), QRT - 1)
                        ldc = jnp.where(inb, ld, QRT)
                        ldv_r[p][pl.ds(g * 16, 16)] = ldc
                        dgat = plsc.load_gather(adv0, [ldg])
                        pre = plsc.load_gather(asv0, [sv]) + dgat
                        alpha = jnp.maximum(pre, 0.2 * pre)
                        m = gm0 + dgat
                        e0 = jnp.exp(alpha - jnp.maximum(m, 0.2 * m))
                        ex0_r[p][pl.ds(g * 16, 16)] = e0
                        plsc.addupdate_scatter(den0, [ldc], e0)
                        if two_heads:
                            dgat1 = plsc.load_gather(adv1, [ldg])
                            pre1 = plsc.load_gather(asv1, [sv]) + dgat1
                            alpha1 = jnp.maximum(pre1, 0.2 * pre1)
                            m1 = gm1 + dgat1
                            e1 = jnp.exp(alpha1 - jnp.maximum(m1, 0.2 * m1))
                            ex1_r[p][pl.ds(g * 16, 16)] = e1
                            plsc.addupdate_scatter(den1, [ldc], e1)
                        srcg_r[p][pl.ds(g * 16, 16)] = sv + row_off
                    pltpu.async_copy(xlp.at[srcg_r[p]], rb_r[p], sem_r[p])

                def process(p):
                    pltpu.make_async_copy(
                        xlp.at[srcg_r[p]], rb_r[p], sem_r[p]).wait()
                    rb = rb_r[p]

                    def krow(k4, kc):
                        for u in range(4):
                            k = k4 * 4 + u
                            k16 = jnp.full((16,), 0, jnp.int32) + k
                            b0 = plsc.load_gather(ex0_r[p], [k16])
                            b1 = (plsc.load_gather(ex1_r[p], [k16])
                                  if two_heads else b0)
                            for q in range(4):
                                rb[k, pl.ds(q * 16, 16)] = (
                                    rb[k, pl.ds(q * 16, 16)] * b0)
                            for q in range(4, 8):
                                rb[k, pl.ds(q * 16, 16)] = (
                                    rb[k, pl.ds(q * 16, 16)] * b1)
                        return kc
                    lax.fori_loop(0, kcl // 4, krow, 0)
                    pltpu.sync_copy(rb, acc_sp.at[ldv_r[p]], add=True)

                def piter(it, carry):
                    for u in range(2):
                        pos = it * 2 + u

                        @pl.when(jnp.logical_and((pos & 7) == 0,
                                                 pos < cnt_s))
                        def _ld():
                            off = pl.multiple_of(
                                lo + (s * per + pos) * kcl, kcl)
                            pltpu.sync_copy(
                                src_e.at[pl.ds(off, 8 * kcl)], srcvb)
                            pltpu.sync_copy(
                                dst_e.at[pl.ds(off, 8 * kcl)], dstvb)

                        @pl.when(pos < cnt_s)
                        def _st():
                            stage(pos & 7, u)

                        @pl.when(jnp.logical_and(pos >= 1,
                                                 pos - 1 < cnt_s))
                        def _pr():
                            process(u ^ 1)
                    return carry
                lax.fori_loop(0, cnt_s // 2 + 1, piter, 0)
                plsc.subcore_barrier()
                drw = QRT // 16
                pltpu.sync_copy(
                    acc_sp.at[pl.ds(pl.multiple_of(s * drw, drw), drw)],
                    acc2.at[pl.ds(
                        pl.multiple_of(row_off + base + s * drw, drw), drw)])
                pltpu.sync_copy(
                    den0.at[pl.ds(0, QRT)],
                    denp.at[pl.ds(pl.multiple_of(
                        ((i * 2) * 16 + s) * NPAD + base, QRT), QRT)])
                if two_heads:
                    pltpu.sync_copy(
                        den1.at[pl.ds(0, QRT)],
                        denp.at[pl.ds(pl.multiple_of(
                            ((i * 2 + 1) * 16 + s) * NPAD + base, QRT),
                            QRT)])
                plsc.subcore_barrier()

    mesh = plsc.VectorSubcoreMesh(core_axis_name="c", subcore_axis_name="s")
    scratch = [pltpu.VMEM((NPAD,), jnp.float32),
               pltpu.VMEM((QRT,), jnp.float32)]
    if two_heads:
        scratch += [pltpu.VMEM((NPAD,), jnp.float32),
                    pltpu.VMEM((QRT,), jnp.float32)]
    scratch += [
        pltpu.VMEM((8 * kcl,), jnp.int32),
        pltpu.VMEM((8 * kcl,), jnp.int32),
        pltpu.VMEM((kcl,), jnp.int32),
        pltpu.VMEM((kcl,), jnp.int32),
        pltpu.VMEM((kcl,), jnp.int32),
        pltpu.VMEM((kcl,), jnp.int32),
        pltpu.VMEM((kcl, 128), jnp.float32),
        pltpu.VMEM((kcl, 128), jnp.float32),
        pltpu.VMEM((kcl,), jnp.float32),
        pltpu.VMEM((kcl,), jnp.float32),
    ]
    if two_heads:
        scratch += [pltpu.VMEM((kcl,), jnp.float32),
                    pltpu.VMEM((kcl,), jnp.float32)]
    scratch += [pltpu.VMEM((DENR,), jnp.float32)]
    if two_heads:
        scratch += [pltpu.VMEM((DENR,), jnp.float32)]
    scratch += [pltpu.VMEM((ZR, 128), jnp.float32),
                pltpu.VMEM((16,), jnp.float32)]
    if two_heads:
        scratch += [pltpu.VMEM((16,), jnp.float32)]
    scratch += [
        pltpu.VMEM((16,), jnp.int32),
        pltpu.SemaphoreType.DMA,
        pltpu.SemaphoreType.DMA,
        pltpu.VMEM_SHARED((ACCR, 128), jnp.float32),
    ]

    return pl.kernel(
        body,
        out_type=(
            jax.ShapeDtypeStruct((npair * NPAD, 128), jnp.float32),
            jax.ShapeDtypeStruct((npair * 2 * 16 * NPAD,), jnp.float32),
        ),
        mesh=mesh,
        scratch_types=scratch,
        compiler_params=pltpu.CompilerParams(needs_layout_passes=False),
    )


def _final_body(acc_ref, dnp_ref, x2_ref, wr_ref, br_ref, bf_ref, out_ref):
    den_s = jnp.sum(dnp_ref[...], axis=2)
    parts = []
    for p in range(2):
        num = jnp.zeros((512, 64), jnp.float32)
        for h in range(HEADS):
            den = den_s[h, 0, :].reshape(512, 1) + 1e-16
            num = num + acc_ref[h, :, p * 64:(p + 1) * 64] / den
        parts.append(num * (1.0 / HEADS) + bf_ref[0, p * 64:(p + 1) * 64])
    h2 = jnp.concatenate(parts, axis=1)
    nrm = jnp.sqrt(jnp.sum(h2 * h2, axis=1, keepdims=True))
    h2 = h2 / jnp.maximum(nrm, 1e-12)
    res = jnp.dot(x2_ref[...], wr_ref[...],
                  preferred_element_type=jnp.float32) + br_ref[0, :][None, :]
    out_ref[...] = h2 + res


def _final_call(accf, denpf, x2, wres, bres, bf):
    x2p = jnp.concatenate(
        [jnp.zeros((N1, IN_CH), jnp.float32), x2,
         jnp.zeros((NPAD - NTOT, IN_CH), jnp.float32)], axis=0)
    full = pl.pallas_call(
        _final_body,
        grid=(NPAD // 512,),
        in_specs=[
            pl.BlockSpec((4, 512, 128), lambda g: (0, g, 0)),
            pl.BlockSpec((4, 2, 16, 512), lambda g: (0, 0, 0, g)),
            pl.BlockSpec((512, IN_CH), lambda g: (g, 0)),
            pl.BlockSpec((IN_CH, IN_CH), lambda g: (0, 0)),
            pl.BlockSpec((1, IN_CH), lambda g: (0, 0)),
            pl.BlockSpec((1, IN_CH), lambda g: (0, 0)),
        ],
        out_specs=pl.BlockSpec((512, IN_CH), lambda g: (g, 0)),
        out_shape=jax.ShapeDtypeStruct((NPAD, IN_CH), jnp.float32),
    )(accf, denpf, x2p, wres, bres.reshape(1, IN_CH), bf.reshape(1, IN_CH))
    return full[N1:NTOT]


def _layer_sc(c_out, xl, ash, adh, gm, src_e, dst_e, bnds):
    npair = HEADS * c_out // 128
    gmx = jnp.broadcast_to(gm.reshape(HEADS, 1), (HEADS, 16)).reshape(-1)
    ash_t = ash.T[:HEADS].reshape(-1)
    adh_t = adh.T[:HEADS].reshape(-1)
    acc2, denp = _make_edge_kernel(npair, c_out)(
        xl.reshape(npair * NPAD, 128), ash_t, adh_t, gmx,
        src_e, dst_e, bnds)
    return (acc2.reshape(npair, NPAD, 128),
            denp.reshape(npair, 2, 16, NPAD))


def kernel(x1, x2, edge_index1, edge_index2, params):
    p = params
    src_e, dst_e, bnds = _edge_prep(edge_index1, edge_index2)
    xc = jnp.concatenate(
        [x1, x2, jnp.zeros((NPAD - NTOT, IN_CH), jnp.float32)], axis=0)

    xl, ash, adh, gm = _pre_call(False, 0, IN_CH, HID, xc,
                                 p['W0'], p['as0'], p['ad0'])
    acc, denp = _layer_sc(HID, xl, ash, adh, gm, src_e, dst_e, bnds)

    xl, ash, adh, gm = _pre_call(True, 2, HID, HID, acc, p['W1'],
                                 p['as1'], p['ad1'], bias=p['b0'], denp=denp)
    acc, denp = _layer_sc(HID, xl, ash, adh, gm, src_e, dst_e, bnds)

    xl, ash, adh, gm = _pre_call(True, 2, HID, IN_CH, acc, p['W2'],
                                 p['as2'], p['ad2'], bias=p['b1'], denp=denp)
    acc, denp = _layer_sc(IN_CH, xl, ash, adh, gm, src_e, dst_e, bnds)

    xl, ash, adh, gm = _pre_call(True, 4, IN_CH, IN_CH, acc, p['Wf'],
                                 p['asf'], p['adf'], bias=p['b2'], denp=denp)
    acc, denp = _layer_sc(IN_CH, xl, ash, adh, gm, src_e, dst_e, bnds)

    return _final_call(acc, denp, x2, p['Wres'], p['bres'], p['bf'])


# speedup vs baseline: 14.5541x; 1.0001x over previous
import functools

import jax
import jax.numpy as jnp
from jax import lax
from jax.experimental import pallas as pl
from jax.experimental.pallas import tpu as pltpu
from jax.experimental.pallas import tpu_sc as plsc

N1 = 10000
N2 = 10000
NTOT = N1 + N2
IN_CH = 128
HID = 64
HEADS = 4

NPAD = 20480
QRT = NPAD // 4
SENT = NTOT * NTOT
ERAW = 3 * 160000 + NTOT
EPAD = 524288
KC = 128
ACCR = QRT + 128
STRIPE = ACCR // 16
ZR = 8
DENR = QRT + 16
NEG = -1e30


def _edge_prep(ei1, ei2):
    ei1 = ei1.astype(jnp.int32)
    ei2 = ei2.astype(jnp.int32)
    src3 = jnp.concatenate([ei1[0], ei2[0] + N1, ei1[0] + N1])
    dst3 = jnp.concatenate([ei1[1], ei2[1] + N1, ei1[1] + N1])
    key = dst3 * NTOT + src3
    key = jnp.where(src3 == dst3, SENT, key)
    ar = jnp.arange(NTOT, dtype=jnp.int32)
    keyd = ar * NTOT + ar
    keys = jnp.sort(jnp.concatenate([key, keyd]))
    dup = jnp.concatenate(
        [jnp.zeros((1,), bool), keys[1:] == keys[:-1]])
    dst = jnp.minimum(keys // NTOT, NTOT)
    src = jnp.where(dup | (keys >= SENT), NTOT, keys % NTOT)
    pad = EPAD + 3072 - ERAW
    src = jnp.concatenate([src, jnp.full((pad,), NTOT, jnp.int32)])
    dst = jnp.concatenate([dst, jnp.full((pad,), NTOT, jnp.int32)])
    los = [jnp.searchsorted(dst, jnp.int32(q * QRT)).astype(jnp.int32)
           for q in range(1, 4)] + [jnp.int32(EPAD)]
    ent = []
    prev_lo = jnp.int32(0)
    for q in range(4):
        lor = (prev_lo // KC) * KC
        nch = (los[q] - lor + KC - 1) // KC
        ent += [lor, nch]
        prev_lo = los[q]
    bnds = jnp.concatenate([jnp.stack(ent), jnp.zeros((8,), jnp.int32)])
    return src, dst, bnds


def _pre_body(has_acc, np_prev, c_prev, npair, c_out, *refs):
    if has_acc:
        (acc_ref, dnp_ref, b_ref, w_ref, as_ref, ad_ref,
         xl_ref, asr_ref, adr_ref, gm_ref) = refs
        den_s = jnp.sum(dnp_ref[...], axis=2)
        parts = []
        for p in range(c_prev // 64):
            num = jnp.zeros((512, 64), jnp.float32)
            for h in range(HEADS):
                j0 = h * c_prev + p * 64
                pair, colo = j0 // 128, j0 % 128
                sub = h % 2 if c_prev == 64 else 0
                den = den_s[pair, sub, :].reshape(512, 1) + 1e-16
                num = num + acc_ref[pair, :, colo:colo + 64] / den
            parts.append(num * (1.0 / HEADS) + b_ref[0, p * 64:(p + 1) * 64])
        xb = jnp.concatenate(parts, axis=1) if len(parts) > 1 else parts[0]
        xb = jnp.maximum(xb, 0.0)
    else:
        (x_ref, w_ref, as_ref, ad_ref,
         xl_ref, asr_ref, adr_ref, gm_ref) = refs
        xb = x_ref[...]
    g = pl.program_id(0)
    y = jnp.dot(xb, w_ref[...], preferred_element_type=jnp.float32)
    rows = lax.broadcasted_iota(jnp.int32, (512, 1), 0) + g * 512
    mask = rows < NTOT
    as_cols, ad_cols, gmx = [], [], []
    for h in range(HEADS):
        yh = y[:, h * c_out:(h + 1) * c_out]
        a_s = jnp.sum(yh * as_ref[h, :][None, :], axis=1, keepdims=True)
        a_d = jnp.sum(yh * ad_ref[h, :][None, :], axis=1, keepdims=True)
        a_s = jnp.where(mask, a_s, NEG)
        a_d = jnp.where(mask, a_d, NEG)
        as_cols.append(a_s)
        ad_cols.append(a_d)
        gmx.append(jnp.max(a_s))
    padc = jnp.full((512, 16 - HEADS), NEG)
    asr_ref[...] = jnp.concatenate(as_cols + [padc], axis=1)
    adr_ref[...] = jnp.concatenate(ad_cols + [padc], axis=1)
    m4 = jnp.stack(gmx).reshape(1, HEADS)
    prev = jnp.where(g == 0, jnp.full((1, HEADS), NEG), gm_ref[...])
    gm_ref[...] = jnp.maximum(prev, m4)
    for q in range(npair):
        xl_ref[q] = y[:, q * 128:(q + 1) * 128]


def _pre_call(has_acc, np_prev, c_prev, c_out, first_arg, w, a_s, a_d,
              bias=None, denp=None):
    npair = HEADS * c_out // 128
    grid = NPAD // 512
    body = functools.partial(_pre_body, has_acc, np_prev, c_prev,
                             npair, c_out)
    in_specs = []
    args = []
    if has_acc:
        in_specs.append(pl.BlockSpec((np_prev, 512, 128),
                                     lambda g: (0, g, 0)))
        args.append(first_arg)
        in_specs.append(pl.BlockSpec((np_prev, 2, 16, 512),
                                     lambda g: (0, 0, 0, g)))
        args.append(denp)
        in_specs.append(pl.BlockSpec((1, c_prev), lambda g: (0, 0)))
        args.append(bias.reshape(1, c_prev))
    else:
        in_specs.append(pl.BlockSpec((512, c_prev), lambda g: (g, 0)))
        args.append(first_arg)
    in_specs += [
        pl.BlockSpec((c_prev, HEADS * c_out), lambda g: (0, 0)),
        pl.BlockSpec((HEADS, c_out), lambda g: (0, 0)),
        pl.BlockSpec((HEADS, c_out), lambda g: (0, 0)),
    ]
    args += [w, a_s, a_d]
    return pl.pallas_call(
        body,
        grid=(grid,),
        in_specs=in_specs,
        out_specs=[
            pl.BlockSpec((npair, 512, 128), lambda g: (0, g, 0)),
            pl.BlockSpec((512, 16), lambda g: (g, 0)),
            pl.BlockSpec((512, 16), lambda g: (g, 0)),
            pl.BlockSpec((1, HEADS), lambda g: (0, 0)),
        ],
        out_shape=[
            jax.ShapeDtypeStruct((npair, NPAD, 128), jnp.float32),
            jax.ShapeDtypeStruct((NPAD, 16), jnp.float32),
            jax.ShapeDtypeStruct((NPAD, 16), jnp.float32),
            jax.ShapeDtypeStruct((1, HEADS), jnp.float32),
        ],
    )(*args)


def _vextract(vec, j):
    return jnp.max(jnp.where(lax.iota(jnp.int32, 16) == j, vec, 0))


@functools.lru_cache(maxsize=None)
def _make_edge_kernel(npair, c_out):
    two_heads = (c_out == 64)
    kcl = 64 if two_heads else 128
    ng = kcl // 16

    def body(xlp, ash, adh, gmx, src_e, dst_e, bnds, acc2, denp, *scr):
        if two_heads:
            (asv0, adv0, asv1, adv1, srcvb, dstvb, srcg0, srcg1,
             ldv0, ldv1, rb0, rb1, ex0a, ex0b, ex1a, ex1b,
             den0, den1, zb, gm0v, gm1v, bndv, sem0, sem1, acc_sp) = scr
        else:
            (asv0, adv0, srcvb, dstvb, srcg0, srcg1,
             ldv0, ldv1, rb0, rb1, ex0a, ex0b,
             den0, zb, gm0v, bndv, sem0, sem1, acc_sp) = scr
            asv1 = adv1 = den1 = gm1v = None
            ex1a = ex1b = None
        srcg_r = (srcg0, srcg1)
        ldv_r = (ldv0, ldv1)
        rb_r = (rb0, rb1)
        ex0_r = (ex0a, ex0b)
        ex1_r = (ex1a, ex1b)
        sem_r = (sem0, sem1)
        c = lax.axis_index("c")
        s = lax.axis_index("s")
        zero16 = jnp.zeros((16,), jnp.float32)

        def zrow(r, carry):
            for q in range(8):
                zb[r, pl.ds(q * 16, 16)] = zero16
            return carry
        lax.fori_loop(0, ZR, zrow, 0)

        pltpu.sync_copy(bnds, bndv)

        for i in range(npair):
            h0 = (i * 128) // c_out
            h1 = (i * 128 + 64) // c_out
            pltpu.sync_copy(gmx.at[pl.ds(h0 * 16, 16)], gm0v)
            pltpu.sync_copy(ash.at[pl.ds(h0 * NPAD, NPAD)], asv0)
            if two_heads:
                pltpu.sync_copy(gmx.at[pl.ds(h1 * 16, 16)], gm1v)
                pltpu.sync_copy(ash.at[pl.ds(h1 * NPAD, NPAD)], asv1)
            gm0 = gm0v[...]
            gm1 = gm1v[...] if two_heads else gm0
            row_off = i * NPAD

            for qq in range(2):
                quarter = qq * 2 + c
                base = pl.multiple_of(quarter * QRT, QRT)
                lo = pl.multiple_of(
                    _vextract(bndv[...], 2 * quarter), KC)
                nchl = _vextract(bndv[...], 2 * quarter + 1) * (KC // kcl)
                per = (nchl + 15) // 16
                cnt_s = jnp.maximum(0, jnp.minimum(per, nchl - s * per))
                pltpu.sync_copy(
                    adh.at[pl.ds(pl.multiple_of(h0 * NPAD + base, QRT),
                                 QRT)], adv0)
                if two_heads:
                    pltpu.sync_copy(
                        adh.at[pl.ds(pl.multiple_of(h1 * NPAD + base, QRT),
                                     QRT)], adv1)
                for z in range(STRIPE // ZR):
                    pltpu.sync_copy(
                        zb, acc_sp.at[pl.ds(s * STRIPE + z * ZR, ZR)])

                def zden(r, carry):
                    den0[pl.ds(r * 16, 16)] = zero16
                    if two_heads:
                        den1[pl.ds(r * 16, 16)] = zero16
                    return carry
                lax.fori_loop(0, DENR // 16, zden, 0)
                plsc.subcore_barrier()

                def stage(bb, p):
                    for g in range(ng):
                        sv = srcvb[pl.ds(bb * kcl + g * 16, 16)]
                        dv = dstvb[pl.ds(bb * kcl + g * 16, 16)]
                        ld = dv - base
                        inb = jnp.logical_and(ld >= 0, ld < QRT)
                        ldg = jnp.minimum(jnp.maximum(ld, 0), QRT - 1)
                        ldc = jnp.where(inb, ld, QRT)
                        ldv_r[p][pl.ds(g * 16, 16)] = ldc
                        dgat = plsc.load_gather(adv0, [ldg])
                        pre = plsc.load_gather(asv0, [sv]) + dgat
                        alpha = jnp.maximum(pre, 0.2 * pre)
                        m = gm0 + dgat
                        e0 = jnp.exp(alpha - jnp.maximum(m, 0.2 * m))
                        ex0_r[p][pl.ds(g * 16, 16)] = e0
                        plsc.addupdate_scatter(den0, [ldc], e0)
                        if two_heads:
                            dgat1 = plsc.load_gather(adv1, [ldg])
                            pre1 = plsc.load_gather(asv1, [sv]) + dgat1
                            alpha1 = jnp.maximum(pre1, 0.2 * pre1)
                            m1 = gm1 + dgat1
                            e1 = jnp.exp(alpha1 - jnp.maximum(m1, 0.2 * m1))
                            ex1_r[p][pl.ds(g * 16, 16)] = e1
                            plsc.addupdate_scatter(den1, [ldc], e1)
                        srcg_r[p][pl.ds(g * 16, 16)] = sv + row_off
                    pltpu.async_copy(xlp.at[srcg_r[p]], rb_r[p], sem_r[p])

                def process(p):
                    pltpu.make_async_copy(
                        xlp.at[srcg_r[p]], rb_r[p], sem_r[p]).wait()
                    rb = rb_r[p]

                    def krow(k4, kc):
                        for u in range(4):
                            k = k4 * 4 + u
                            k16 = jnp.full((16,), 0, jnp.int32) + k
                            b0 = plsc.load_gather(ex0_r[p], [k16])
                            b1 = (plsc.load_gather(ex1_r[p], [k16])
                                  if two_heads else b0)
                            for q in range(4):
                                rb[k, pl.ds(q * 16, 16)] = (
                                    rb[k, pl.ds(q * 16, 16)] * b0)
                            for q in range(4, 8):
                                rb[k, pl.ds(q * 16, 16)] = (
                                    rb[k, pl.ds(q * 16, 16)] * b1)
                        return kc
                    lax.fori_loop(0, kcl // 4, krow, 0)
                    pltpu.sync_copy(rb, acc_sp.at[ldv_r[p]], add=True)

                def piter(it, carry):
                    for u in range(2):
                        pos = it * 2 + u

                        @pl.when(jnp.logical_and((pos & 7) == 0,
                                                 pos < cnt_s))
                        def _ld():
                            off = pl.multiple_of(
                                lo + (s * per + pos) * kcl, kcl)
                            pltpu.sync_copy(
                                src_e.at[pl.ds(off, 8 * kcl)], srcvb)
                            pltpu.sync_copy(
                                dst_e.at[pl.ds(off, 8 * kcl)], dstvb)

                        @pl.when(pos < cnt_s)
                        def _st():
                            stage(pos & 7, u)

                        @pl.when(jnp.logical_and(pos >= 1,
                                                 pos - 1 < cnt_s))
                        def _pr():
                            process(u ^ 1)
                    return carry
                lax.fori_loop(0, cnt_s // 2 + 1, piter, 0)
                plsc.subcore_barrier()
                drw = QRT // 16
                pltpu.sync_copy(
                    acc_sp.at[pl.ds(pl.multiple_of(s * drw, drw), drw)],
                    acc2.at[pl.ds(
                        pl.multiple_of(row_off + base + s * drw, drw), drw)])
                pltpu.sync_copy(
                    den0.at[pl.ds(0, QRT)],
                    denp.at[pl.ds(pl.multiple_of(
                        ((i * 2) * 16 + s) * NPAD + base, QRT), QRT)])
                if two_heads:
                    pltpu.sync_copy(
                        den1.at[pl.ds(0, QRT)],
                        denp.at[pl.ds(pl.multiple_of(
                            ((i * 2 + 1) * 16 + s) * NPAD + base, QRT),
                            QRT)])
                plsc.subcore_barrier()

    mesh = plsc.VectorSubcoreMesh(core_axis_name="c", subcore_axis_name="s")
    scratch = [pltpu.VMEM((NPAD,), jnp.float32),
               pltpu.VMEM((QRT,), jnp.float32)]
    if two_heads:
        scratch += [pltpu.VMEM((NPAD,), jnp.float32),
                    pltpu.VMEM((QRT,), jnp.float32)]
    scratch += [
        pltpu.VMEM((8 * kcl,), jnp.int32),
        pltpu.VMEM((8 * kcl,), jnp.int32),
        pltpu.VMEM((kcl,), jnp.int32),
        pltpu.VMEM((kcl,), jnp.int32),
        pltpu.VMEM((kcl,), jnp.int32),
        pltpu.VMEM((kcl,), jnp.int32),
        pltpu.VMEM((kcl, 128), jnp.float32),
        pltpu.VMEM((kcl, 128), jnp.float32),
        pltpu.VMEM((kcl,), jnp.float32),
        pltpu.VMEM((kcl,), jnp.float32),
    ]
    if two_heads:
        scratch += [pltpu.VMEM((kcl,), jnp.float32),
                    pltpu.VMEM((kcl,), jnp.float32)]
    scratch += [pltpu.VMEM((DENR,), jnp.float32)]
    if two_heads:
        scratch += [pltpu.VMEM((DENR,), jnp.float32)]
    scratch += [pltpu.VMEM((ZR, 128), jnp.float32),
                pltpu.VMEM((16,), jnp.float32)]
    if two_heads:
        scratch += [pltpu.VMEM((16,), jnp.float32)]
    scratch += [
        pltpu.VMEM((16,), jnp.int32),
        pltpu.SemaphoreType.DMA,
        pltpu.SemaphoreType.DMA,
        pltpu.VMEM_SHARED((ACCR, 128), jnp.float32),
    ]

    return pl.kernel(
        body,
        out_type=(
            jax.ShapeDtypeStruct((npair * NPAD, 128), jnp.float32),
            jax.ShapeDtypeStruct((npair * 2 * 16 * NPAD,), jnp.float32),
        ),
        mesh=mesh,
        scratch_types=scratch,
        compiler_params=pltpu.CompilerParams(needs_layout_passes=False),
    )


def _final_body(acc_ref, dnp_ref, x2_ref, wr_ref, br_ref, bf_ref, out_ref):
    den_s = jnp.sum(dnp_ref[...], axis=2)
    parts = []
    for p in range(2):
        num = jnp.zeros((512, 64), jnp.float32)
        for h in range(HEADS):
            den = den_s[h, 0, :].reshape(512, 1) + 1e-16
            num = num + acc_ref[h, :, p * 64:(p + 1) * 64] / den
        parts.append(num * (1.0 / HEADS) + bf_ref[0, p * 64:(p + 1) * 64])
    h2 = jnp.concatenate(parts, axis=1)
    nrm = jnp.sqrt(jnp.sum(h2 * h2, axis=1, keepdims=True))
    h2 = h2 / jnp.maximum(nrm, 1e-12)
    res = jnp.dot(x2_ref[...], wr_ref[...],
                  preferred_element_type=jnp.float32) + br_ref[0, :][None, :]
    out_ref[...] = h2 + res


def _final_call(accf, denpf, x2, wres, bres, bf):
    x2p = jnp.concatenate(
        [jnp.zeros((N1, IN_CH), jnp.float32), x2,
         jnp.zeros((NPAD - NTOT, IN_CH), jnp.float32)], axis=0)
    full = pl.pallas_call(
        _final_body,
        grid=(NPAD // 512,),
        in_specs=[
            pl.BlockSpec((4, 512, 128), lambda g: (0, g, 0)),
            pl.BlockSpec((4, 2, 16, 512), lambda g: (0, 0, 0, g)),
            pl.BlockSpec((512, IN_CH), lambda g: (g, 0)),
            pl.BlockSpec((IN_CH, IN_CH), lambda g: (0, 0)),
            pl.BlockSpec((1, IN_CH), lambda g: (0, 0)),
            pl.BlockSpec((1, IN_CH), lambda g: (0, 0)),
        ],
        out_specs=pl.BlockSpec((512, IN_CH), lambda g: (g, 0)),
        out_shape=jax.ShapeDtypeStruct((NPAD, IN_CH), jnp.float32),
    )(accf, denpf, x2p, wres, bres.reshape(1, IN_CH), bf.reshape(1, IN_CH))
    return full[N1:NTOT]


def _layer_sc(c_out, xl, ash, adh, gm, src_e, dst_e, bnds):
    npair = HEADS * c_out // 128
    gmx = jnp.broadcast_to(gm.reshape(HEADS, 1), (HEADS, 16)).reshape(-1)
    ash_t = ash.T[:HEADS].reshape(-1)
    adh_t = adh.T[:HEADS].reshape(-1)
    acc2, denp = _make_edge_kernel(npair, c_out)(
        xl.reshape(npair * NPAD, 128), ash_t, adh_t, gmx,
        src_e, dst_e, bnds)
    return (acc2.reshape(npair, NPAD, 128),
            denp.reshape(npair, 2, 16, NPAD))


def kernel(x1, x2, edge_index1, edge_index2, params):
    p = params
    src_e, dst_e, bnds = _edge_prep(edge_index1, edge_index2)
    xc = jnp.concatenate(
        [x1, x2, jnp.zeros((NPAD - NTOT, IN_CH), jnp.float32)], axis=0)

    xl, ash, adh, gm = _pre_call(False, 0, IN_CH, HID, xc,
                                 p['W0'], p['as0'], p['ad0'])
    acc, denp = _layer_sc(HID, xl, ash, adh, gm, src_e, dst_e, bnds)

    xl, ash, adh, gm = _pre_call(True, 2, HID, HID, acc, p['W1'],
                                 p['as1'], p['ad1'], bias=p['b0'], denp=denp)
    acc, denp = _layer_sc(HID, xl, ash, adh, gm, src_e, dst_e, bnds)

    xl, ash, adh, gm = _pre_call(True, 2, HID, IN_CH, acc, p['W2'],
                                 p['as2'], p['ad2'], bias=p['b1'], denp=denp)
    acc, denp = _layer_sc(IN_CH, xl, ash, adh, gm, src_e, dst_e, bnds)

    xl, ash, adh, gm = _pre_call(True, 4, IN_CH, IN_CH, acc, p['Wf'],
                                 p['asf'], p['adf'], bias=p['b2'], denp=denp)
    acc, denp = _layer_sc(IN_CH, xl, ash, adh, gm, src_e, dst_e, bnds)

    return _final_call(acc, denp, x2, p['Wres'], p['bres'], p['bf'])


# speedup vs baseline: 14.5859x; 1.0022x over previous
import functools

import jax
import jax.numpy as jnp
from jax import lax
from jax.experimental import pallas as pl
from jax.experimental.pallas import tpu as pltpu
from jax.experimental.pallas import tpu_sc as plsc

N1 = 10000
N2 = 10000
NTOT = N1 + N2
IN_CH = 128
HID = 64
HEADS = 4

NPAD = 20480
QRT = NPAD // 4
SENT = NTOT * NTOT
ERAW = 3 * 160000 + NTOT
EPAD = 524288
KC = 128
ACCR = QRT + 128
STRIPE = ACCR // 16
ZR = 8
DENR = QRT + 16
NEG = -1e30


def _edge_prep(ei1, ei2):
    ei1 = ei1.astype(jnp.int32)
    ei2 = ei2.astype(jnp.int32)
    src3 = jnp.concatenate([ei1[0], ei2[0] + N1, ei1[0] + N1])
    dst3 = jnp.concatenate([ei1[1], ei2[1] + N1, ei1[1] + N1])
    key = dst3 * NTOT + src3
    key = jnp.where(src3 == dst3, SENT, key)
    ar = jnp.arange(NTOT, dtype=jnp.int32)
    keyd = ar * NTOT + ar
    keys = jnp.sort(jnp.concatenate([key, keyd]))
    dup = jnp.concatenate(
        [jnp.zeros((1,), bool), keys[1:] == keys[:-1]])
    dst = jnp.minimum(keys // NTOT, NTOT)
    src = jnp.where(dup | (keys >= SENT), NTOT, keys % NTOT)
    pad = EPAD + 3072 - ERAW
    src = jnp.concatenate([src, jnp.full((pad,), NTOT, jnp.int32)])
    dst = jnp.concatenate([dst, jnp.full((pad,), NTOT, jnp.int32)])
    los = [jnp.searchsorted(dst, jnp.int32(q * QRT)).astype(jnp.int32)
           for q in range(1, 4)] + [jnp.int32(EPAD)]
    ent = []
    prev_lo = jnp.int32(0)
    for q in range(4):
        lor = (prev_lo // KC) * KC
        nch = (los[q] - lor + KC - 1) // KC
        ent += [lor, nch]
        prev_lo = los[q]
    bnds = jnp.concatenate([jnp.stack(ent), jnp.zeros((8,), jnp.int32)])
    return src, dst, bnds


def _pre_body(has_acc, np_prev, c_prev, npair, c_out, *refs):
    if has_acc:
        (acc_ref, dnp_ref, b_ref, w_ref, as_ref, ad_ref,
         xl_ref, asr_ref, adr_ref, gm_ref) = refs
        den_s = jnp.sum(dnp_ref[...], axis=2)
        parts = []
        for p in range(c_prev // 64):
            num = jnp.zeros((512, 64), jnp.float32)
            for h in range(HEADS):
                j0 = h * c_prev + p * 64
                pair, colo = j0 // 128, j0 % 128
                sub = h % 2 if c_prev == 64 else 0
                den = den_s[pair, sub, :].reshape(512, 1) + 1e-16
                num = num + acc_ref[pair, :, colo:colo + 64] / den
            parts.append(num * (1.0 / HEADS) + b_ref[0, p * 64:(p + 1) * 64])
        xb = jnp.concatenate(parts, axis=1) if len(parts) > 1 else parts[0]
        xb = jnp.maximum(xb, 0.0)
    else:
        (x_ref, w_ref, as_ref, ad_ref,
         xl_ref, asr_ref, adr_ref, gm_ref) = refs
        xb = x_ref[...]
    g = pl.program_id(0)
    y = jnp.dot(xb, w_ref[...], preferred_element_type=jnp.float32)
    rows = lax.broadcasted_iota(jnp.int32, (512, 1), 0) + g * 512
    mask = rows < NTOT
    as_cols, ad_cols, gmx = [], [], []
    for h in range(HEADS):
        yh = y[:, h * c_out:(h + 1) * c_out]
        a_s = jnp.sum(yh * as_ref[h, :][None, :], axis=1, keepdims=True)
        a_d = jnp.sum(yh * ad_ref[h, :][None, :], axis=1, keepdims=True)
        a_s = jnp.where(mask, a_s, NEG)
        a_d = jnp.where(mask, a_d, NEG)
        as_cols.append(a_s)
        ad_cols.append(a_d)
        gmx.append(jnp.max(a_s))
    padc = jnp.full((512, 16 - HEADS), NEG)
    asr_ref[...] = jnp.concatenate(as_cols + [padc], axis=1)
    adr_ref[...] = jnp.concatenate(ad_cols + [padc], axis=1)
    m4 = jnp.stack(gmx).reshape(1, HEADS)
    prev = jnp.where(g == 0, jnp.full((1, HEADS), NEG), gm_ref[...])
    gm_ref[...] = jnp.maximum(prev, m4)
    for q in range(npair):
        xl_ref[q] = y[:, q * 128:(q + 1) * 128]


def _pre_call(has_acc, np_prev, c_prev, c_out, first_arg, w, a_s, a_d,
              bias=None, denp=None):
    npair = HEADS * c_out // 128
    grid = NPAD // 512
    body = functools.partial(_pre_body, has_acc, np_prev, c_prev,
                             npair, c_out)
    in_specs = []
    args = []
    if has_acc:
        in_specs.append(pl.BlockSpec((np_prev, 512, 128),
                                     lambda g: (0, g, 0)))
        args.append(first_arg)
        in_specs.append(pl.BlockSpec((np_prev, 2, 16, 512),
                                     lambda g: (0, 0, 0, g)))
        args.append(denp)
        in_specs.append(pl.BlockSpec((1, c_prev), lambda g: (0, 0)))
        args.append(bias.reshape(1, c_prev))
    else:
        in_specs.append(pl.BlockSpec((512, c_prev), lambda g: (g, 0)))
        args.append(first_arg)
    in_specs += [
        pl.BlockSpec((c_prev, HEADS * c_out), lambda g: (0, 0)),
        pl.BlockSpec((HEADS, c_out), lambda g: (0, 0)),
        pl.BlockSpec((HEADS, c_out), lambda g: (0, 0)),
    ]
    args += [w, a_s, a_d]
    return pl.pallas_call(
        body,
        grid=(grid,),
        in_specs=in_specs,
        out_specs=[
            pl.BlockSpec((npair, 512, 128), lambda g: (0, g, 0)),
            pl.BlockSpec((512, 16), lambda g: (g, 0)),
            pl.BlockSpec((512, 16), lambda g: (g, 0)),
            pl.BlockSpec((1, HEADS), lambda g: (0, 0)),
        ],
        out_shape=[
            jax.ShapeDtypeStruct((npair, NPAD, 128), jnp.float32),
            jax.ShapeDtypeStruct((NPAD, 16), jnp.float32),
            jax.ShapeDtypeStruct((NPAD, 16), jnp.float32),
            jax.ShapeDtypeStruct((1, HEADS), jnp.float32),
        ],
    )(*args)


def _vextract(vec, j):
    return jnp.max(jnp.where(lax.iota(jnp.int32, 16) == j, vec, 0))


@functools.lru_cache(maxsize=None)
def _make_edge_kernel(npair, c_out):
    two_heads = (c_out == 64)
    kcl = 64 if two_heads else 128
    ng = kcl // 16

    def body(xlp, ash, adh, gmx, src_e, dst_e, bnds, acc2, denp, *scr):
        if two_heads:
            (asv0, adv0, asv1, adv1, srcvb, dstvb, srcg0, srcg1,
             ldv0, ldv1, rb0, rb1, ex0a, ex0b, ex1a, ex1b,
             den0, den1, zb, gm0v, gm1v, bndv, sem0, sem1,
             semsc0, semsc1, acc_sp) = scr
        else:
            (asv0, adv0, srcvb, dstvb, srcg0, srcg1,
             ldv0, ldv1, rb0, rb1, ex0a, ex0b,
             den0, zb, gm0v, bndv, sem0, sem1,
             semsc0, semsc1, acc_sp) = scr
            asv1 = adv1 = den1 = gm1v = None
            ex1a = ex1b = None
        sem_sc = (semsc0, semsc1)
        srcg_r = (srcg0, srcg1)
        ldv_r = (ldv0, ldv1)
        rb_r = (rb0, rb1)
        ex0_r = (ex0a, ex0b)
        ex1_r = (ex1a, ex1b)
        sem_r = (sem0, sem1)
        c = lax.axis_index("c")
        s = lax.axis_index("s")
        zero16 = jnp.zeros((16,), jnp.float32)

        def zrow(r, carry):
            for q in range(8):
                zb[r, pl.ds(q * 16, 16)] = zero16
            return carry
        lax.fori_loop(0, ZR, zrow, 0)

        pltpu.sync_copy(bnds, bndv)

        for i in range(npair):
            h0 = (i * 128) // c_out
            h1 = (i * 128 + 64) // c_out
            pltpu.sync_copy(gmx.at[pl.ds(h0 * 16, 16)], gm0v)
            pltpu.sync_copy(ash.at[pl.ds(h0 * NPAD, NPAD)], asv0)
            if two_heads:
                pltpu.sync_copy(gmx.at[pl.ds(h1 * 16, 16)], gm1v)
                pltpu.sync_copy(ash.at[pl.ds(h1 * NPAD, NPAD)], asv1)
            gm0 = gm0v[...]
            gm1 = gm1v[...] if two_heads else gm0
            row_off = i * NPAD

            for qq in range(2):
                quarter = qq * 2 + c
                base = pl.multiple_of(quarter * QRT, QRT)
                lo = pl.multiple_of(
                    _vextract(bndv[...], 2 * quarter), KC)
                nchl = _vextract(bndv[...], 2 * quarter + 1) * (KC // kcl)
                per = (nchl + 15) // 16
                cnt_s = jnp.maximum(0, jnp.minimum(per, nchl - s * per))
                pltpu.sync_copy(
                    adh.at[pl.ds(pl.multiple_of(h0 * NPAD + base, QRT),
                                 QRT)], adv0)
                if two_heads:
                    pltpu.sync_copy(
                        adh.at[pl.ds(pl.multiple_of(h1 * NPAD + base, QRT),
                                     QRT)], adv1)
                for z in range(STRIPE // ZR):
                    pltpu.sync_copy(
                        zb, acc_sp.at[pl.ds(s * STRIPE + z * ZR, ZR)])

                def zden(r, carry):
                    den0[pl.ds(r * 16, 16)] = zero16
                    if two_heads:
                        den1[pl.ds(r * 16, 16)] = zero16
                    return carry
                lax.fori_loop(0, DENR // 16, zden, 0)
                plsc.subcore_barrier()

                def stage(bb, p, pos):
                    @pl.when(pos >= 2)
                    def _wsc():
                        pltpu.make_async_copy(
                            rb_r[p], acc_sp.at[ldv_r[p]], sem_sc[p]).wait()
                    for g in range(ng):
                        sv = srcvb[pl.ds(bb * kcl + g * 16, 16)]
                        dv = dstvb[pl.ds(bb * kcl + g * 16, 16)]
                        ld = dv - base
                        inb = jnp.logical_and(ld >= 0, ld < QRT)
                        ldg = jnp.minimum(jnp.maximum(ld, 0), QRT - 1)
                        ldc = jnp.where(inb, ld, QRT)
                        ldv_r[p][pl.ds(g * 16, 16)] = ldc
                        dgat = plsc.load_gather(adv0, [ldg])
                        pre = plsc.load_gather(asv0, [sv]) + dgat
                        alpha = jnp.maximum(pre, 0.2 * pre)
                        m = gm0 + dgat
                        e0 = jnp.exp(alpha - jnp.maximum(m, 0.2 * m))
                        ex0_r[p][pl.ds(g * 16, 16)] = e0
                        plsc.addupdate_scatter(den0, [ldc], e0)
                        if two_heads:
                            dgat1 = plsc.load_gather(adv1, [ldg])
                            pre1 = plsc.load_gather(asv1, [sv]) + dgat1
                            alpha1 = jnp.maximum(pre1, 0.2 * pre1)
                            m1 = gm1 + dgat1
                            e1 = jnp.exp(alpha1 - jnp.maximum(m1, 0.2 * m1))
                            ex1_r[p][pl.ds(g * 16, 16)] = e1
                            plsc.addupdate_scatter(den1, [ldc], e1)
                        srcg_r[p][pl.ds(g * 16, 16)] = sv + row_off
                    pltpu.async_copy(xlp.at[srcg_r[p]], rb_r[p], sem_r[p])

                def process(p):
                    pltpu.make_async_copy(
                        xlp.at[srcg_r[p]], rb_r[p], sem_r[p]).wait()
                    rb = rb_r[p]

                    def krow(k4, kc):
                        for u in range(4):
                            k = k4 * 4 + u
                            k16 = jnp.full((16,), 0, jnp.int32) + k
                            b0 = plsc.load_gather(ex0_r[p], [k16])
                            b1 = (plsc.load_gather(ex1_r[p], [k16])
                                  if two_heads else b0)
                            for q in range(4):
                                rb[k, pl.ds(q * 16, 16)] = (
                                    rb[k, pl.ds(q * 16, 16)] * b0)
                            for q in range(4, 8):
                                rb[k, pl.ds(q * 16, 16)] = (
                                    rb[k, pl.ds(q * 16, 16)] * b1)
                        return kc
                    lax.fori_loop(0, kcl // 4, krow, 0)
                    pltpu.async_copy(rb, acc_sp.at[ldv_r[p]], sem_sc[p],
                                     add=True)

                def piter(it, carry):
                    for u in range(2):
                        pos = it * 2 + u

                        @pl.when(jnp.logical_and((pos & 7) == 0,
                                                 pos < cnt_s))
                        def _ld():
                            off = pl.multiple_of(
                                lo + (s * per + pos) * kcl, kcl)
                            pltpu.sync_copy(
                                src_e.at[pl.ds(off, 8 * kcl)], srcvb)
                            pltpu.sync_copy(
                                dst_e.at[pl.ds(off, 8 * kcl)], dstvb)

                        @pl.when(pos < cnt_s)
                        def _st():
                            stage(pos & 7, u, pos)

                        @pl.when(jnp.logical_and(pos >= 1,
                                                 pos - 1 < cnt_s))
                        def _pr():
                            process(u ^ 1)
                    return carry
                lax.fori_loop(0, cnt_s // 2 + 1, piter, 0)
                for p in range(2):
                    @pl.when(jnp.logical_and(cnt_s > 0,
                                             (cnt_s - 1) & 1 == p))
                    def _dr1():
                        pltpu.make_async_copy(
                            rb_r[p], acc_sp.at[ldv_r[p]], sem_sc[p]).wait()

                    @pl.when(jnp.logical_and(cnt_s > 1,
                                             (cnt_s - 2) & 1 == p))
                    def _dr2():
                        pltpu.make_async_copy(
                            rb_r[p], acc_sp.at[ldv_r[p]], sem_sc[p]).wait()
                plsc.subcore_barrier()
                drw = QRT // 16
                pltpu.sync_copy(
                    acc_sp.at[pl.ds(pl.multiple_of(s * drw, drw), drw)],
                    acc2.at[pl.ds(
                        pl.multiple_of(row_off + base + s * drw, drw), drw)])
                pltpu.sync_copy(
                    den0.at[pl.ds(0, QRT)],
                    denp.at[pl.ds(pl.multiple_of(
                        ((i * 2) * 16 + s) * NPAD + base, QRT), QRT)])
                if two_heads:
                    pltpu.sync_copy(
                        den1.at[pl.ds(0, QRT)],
                        denp.at[pl.ds(pl.multiple_of(
                            ((i * 2 + 1) * 16 + s) * NPAD + base, QRT),
                            QRT)])
                plsc.subcore_barrier()

    mesh = plsc.VectorSubcoreMesh(core_axis_name="c", subcore_axis_name="s")
    scratch = [pltpu.VMEM((NPAD,), jnp.float32),
               pltpu.VMEM((QRT,), jnp.float32)]
    if two_heads:
        scratch += [pltpu.VMEM((NPAD,), jnp.float32),
                    pltpu.VMEM((QRT,), jnp.float32)]
    scratch += [
        pltpu.VMEM((8 * kcl,), jnp.int32),
        pltpu.VMEM((8 * kcl,), jnp.int32),
        pltpu.VMEM((kcl,), jnp.int32),
        pltpu.VMEM((kcl,), jnp.int32),
        pltpu.VMEM((kcl,), jnp.int32),
        pltpu.VMEM((kcl,), jnp.int32),
        pltpu.VMEM((kcl, 128), jnp.float32),
        pltpu.VMEM((kcl, 128), jnp.float32),
        pltpu.VMEM((kcl,), jnp.float32),
        pltpu.VMEM((kcl,), jnp.float32),
    ]
    if two_heads:
        scratch += [pltpu.VMEM((kcl,), jnp.float32),
                    pltpu.VMEM((kcl,), jnp.float32)]
    scratch += [pltpu.VMEM((DENR,), jnp.float32)]
    if two_heads:
        scratch += [pltpu.VMEM((DENR,), jnp.float32)]
    scratch += [pltpu.VMEM((ZR, 128), jnp.float32),
                pltpu.VMEM((16,), jnp.float32)]
    if two_heads:
        scratch += [pltpu.VMEM((16,), jnp.float32)]
    scratch += [
        pltpu.VMEM((16,), jnp.int32),
        pltpu.SemaphoreType.DMA,
        pltpu.SemaphoreType.DMA,
        pltpu.SemaphoreType.DMA,
        pltpu.SemaphoreType.DMA,
        pltpu.VMEM_SHARED((ACCR, 128), jnp.float32),
    ]

    return pl.kernel(
        body,
        out_type=(
            jax.ShapeDtypeStruct((npair * NPAD, 128), jnp.float32),
            jax.ShapeDtypeStruct((npair * 2 * 16 * NPAD,), jnp.float32),
        ),
        mesh=mesh,
        scratch_types=scratch,
        compiler_params=pltpu.CompilerParams(needs_layout_passes=False),
    )


def _final_body(acc_ref, dnp_ref, x2_ref, wr_ref, br_ref, bf_ref, out_ref):
    den_s = jnp.sum(dnp_ref[...], axis=2)
    parts = []
    for p in range(2):
        num = jnp.zeros((512, 64), jnp.float32)
        for h in range(HEADS):
            den = den_s[h, 0, :].reshape(512, 1) + 1e-16
            num = num + acc_ref[h, :, p * 64:(p + 1) * 64] / den
        parts.append(num * (1.0 / HEADS) + bf_ref[0, p * 64:(p + 1) * 64])
    h2 = jnp.concatenate(parts, axis=1)
    nrm = jnp.sqrt(jnp.sum(h2 * h2, axis=1, keepdims=True))
    h2 = h2 / jnp.maximum(nrm, 1e-12)
    res = jnp.dot(x2_ref[...], wr_ref[...],
                  preferred_element_type=jnp.float32) + br_ref[0, :][None, :]
    out_ref[...] = h2 + res


def _final_call(accf, denpf, x2, wres, bres, bf):
    x2p = jnp.concatenate(
        [jnp.zeros((N1, IN_CH), jnp.float32), x2,
         jnp.zeros((NPAD - NTOT, IN_CH), jnp.float32)], axis=0)
    full = pl.pallas_call(
        _final_body,
        grid=(NPAD // 512,),
        in_specs=[
            pl.BlockSpec((4, 512, 128), lambda g: (0, g, 0)),
            pl.BlockSpec((4, 2, 16, 512), lambda g: (0, 0, 0, g)),
            pl.BlockSpec((512, IN_CH), lambda g: (g, 0)),
            pl.BlockSpec((IN_CH, IN_CH), lambda g: (0, 0)),
            pl.BlockSpec((1, IN_CH), lambda g: (0, 0)),
            pl.BlockSpec((1, IN_CH), lambda g: (0, 0)),
        ],
        out_specs=pl.BlockSpec((512, IN_CH), lambda g: (g, 0)),
        out_shape=jax.ShapeDtypeStruct((NPAD, IN_CH), jnp.float32),
    )(accf, denpf, x2p, wres, bres.reshape(1, IN_CH), bf.reshape(1, IN_CH))
    return full[N1:NTOT]


def _layer_sc(c_out, xl, ash, adh, gm, src_e, dst_e, bnds):
    npair = HEADS * c_out // 128
    gmx = jnp.broadcast_to(gm.reshape(HEADS, 1), (HEADS, 16)).reshape(-1)
    ash_t = ash.T[:HEADS].reshape(-1)
    adh_t = adh.T[:HEADS].reshape(-1)
    acc2, denp = _make_edge_kernel(npair, c_out)(
        xl.reshape(npair * NPAD, 128), ash_t, adh_t, gmx,
        src_e, dst_e, bnds)
    return (acc2.reshape(npair, NPAD, 128),
            denp.reshape(npair, 2, 16, NPAD))


def kernel(x1, x2, edge_index1, edge_index2, params):
    p = params
    src_e, dst_e, bnds = _edge_prep(edge_index1, edge_index2)
    xc = jnp.concatenate(
        [x1, x2, jnp.zeros((NPAD - NTOT, IN_CH), jnp.float32)], axis=0)

    xl, ash, adh, gm = _pre_call(False, 0, IN_CH, HID, xc,
                                 p['W0'], p['as0'], p['ad0'])
    acc, denp = _layer_sc(HID, xl, ash, adh, gm, src_e, dst_e, bnds)

    xl, ash, adh, gm = _pre_call(True, 2, HID, HID, acc, p['W1'],
                                 p['as1'], p['ad1'], bias=p['b0'], denp=denp)
    acc, denp = _layer_sc(HID, xl, ash, adh, gm, src_e, dst_e, bnds)

    xl, ash, adh, gm = _pre_call(True, 2, HID, IN_CH, acc, p['W2'],
                                 p['as2'], p['ad2'], bias=p['b1'], denp=denp)
    acc, denp = _layer_sc(IN_CH, xl, ash, adh, gm, src_e, dst_e, bnds)

    xl, ash, adh, gm = _pre_call(True, 4, IN_CH, IN_CH, acc, p['Wf'],
                                 p['asf'], p['adf'], bias=p['b2'], denp=denp)
    acc, denp = _layer_sc(IN_CH, xl, ash, adh, gm, src_e, dst_e, bnds)

    return _final_call(acc, denp, x2, p['Wres'], p['bres'], p['bf'])
